# initial kernel scaffold (unmeasured)
import jax
import jax.numpy as jnp
from jax import lax
from jax.experimental import pallas as pl
from jax.experimental.pallas import tpu as pltpu


def kernel(
    x,
):
    def body(*refs):
        pass

    out_shape = jax.ShapeDtypeStruct(..., jnp.float32)
    return pl.pallas_call(body, out_shape=out_shape)(...)


# baseline (device time: 10610 ns/iter reference)
import jax
import jax.numpy as jnp
from jax import lax
from jax.experimental import pallas as pl
from jax.experimental.pallas import tpu as pltpu

N_DEV = 16


def _local_cumprod(a, m, n):
    k = 1
    while k < m:
        shifted = jnp.concatenate(
            [jnp.ones((k, n), a.dtype), a[: m - k, :]], axis=0
        )
        a = a * shifted
        k *= 2
    return a


def _tree_prod(a, m):
    while m > 1:
        m //= 2
        a = a[:m, :] * a[m : 2 * m, :]
    return a


def kernel(x):
    m, n = x.shape

    def body(x_ref, out_ref, send_buf, slots, send_sems, recv_sems):
        my = lax.axis_index("i")

        a = x_ref[:, :].astype(jnp.float32)
        send_buf[0, :] = _tree_prod(a, m)[0, :]

        rdmas = []
        for d in range(1, N_DEV):
            rdma = pltpu.make_async_remote_copy(
                src_ref=send_buf,
                dst_ref=slots.at[pl.ds(d - 1, 1)],
                send_sem=send_sems.at[d - 1],
                recv_sem=recv_sems.at[d - 1],
                device_id=((my + d) % N_DEV,),
                device_id_type=pl.DeviceIdType.MESH,
            )
            rdmas.append(rdma)

            @pl.when(my + d < N_DEV)
            def _(rdma=rdma):
                rdma.start()

        scan = _local_cumprod(a, m, n)

        for d in range(1, N_DEV):

            @pl.when(d <= my)
            def _(rdma=rdmas[d - 1]):
                rdma.wait_recv()

        vals = slots[:, :]
        idx = lax.broadcasted_iota(jnp.int32, (N_DEV - 1, n), 0)
        vals = jnp.where(idx < my, vals, jnp.ones_like(vals))
        ones_row = jnp.ones((1, n), jnp.float32)
        prefix = _tree_prod(jnp.concatenate([vals, ones_row], axis=0), N_DEV)

        out_ref[:, :] = scan * prefix

        for d in range(1, N_DEV):

            @pl.when(my + d < N_DEV)
            def _(rdma=rdmas[d - 1]):
                rdma.wait_send()

    return pl.pallas_call(
        body,
        out_shape=jax.ShapeDtypeStruct((m, n), jnp.float32),
        in_specs=[pl.BlockSpec(memory_space=pltpu.VMEM)],
        out_specs=pl.BlockSpec(memory_space=pltpu.VMEM),
        scratch_shapes=[
            pltpu.VMEM((1, n), jnp.float32),
            pltpu.VMEM((N_DEV - 1, n), jnp.float32),
            pltpu.SemaphoreType.DMA((N_DEV - 1,)),
            pltpu.SemaphoreType.DMA((N_DEV - 1,)),
        ],
    )(x)


# device time: 9167 ns/iter; 1.1574x vs baseline; 1.1574x over previous
import jax
import jax.numpy as jnp
from jax import lax
from jax.experimental import pallas as pl
from jax.experimental.pallas import tpu as pltpu

N_DEV = 16


def _local_cumprod(a, m, n):
    k = 1
    while k < m:
        shifted = jnp.concatenate(
            [jnp.ones((k, n), a.dtype), a[: m - k, :]], axis=0
        )
        a = a * shifted
        k *= 2
    return a


def _tree_prod(a, m):
    while m > 1:
        m //= 2
        a = a[:m, :] * a[m : 2 * m, :]
    return a


def kernel(x):
    m, n = x.shape

    def body(x_ref, out_ref, send_buf, slots, send_sems, recv_sems):
        my = lax.axis_index("i")

        barrier_sem = pltpu.get_barrier_semaphore()
        for r in range(1, N_DEV):

            @pl.when(r <= my)
            def _(r=r):
                pl.semaphore_signal(
                    barrier_sem,
                    inc=1,
                    device_id=(my - r,),
                    device_id_type=pl.DeviceIdType.MESH,
                )

        a = x_ref[:, :].astype(jnp.float32)
        send_buf[0, :] = _tree_prod(a, m)[0, :]

        for r in range(1, N_DEV):

            @pl.when(my + r < N_DEV)
            def _():
                pl.semaphore_wait(barrier_sem, 1)

        rdmas = []
        for d in range(1, N_DEV):
            rdma = pltpu.make_async_remote_copy(
                src_ref=send_buf,
                dst_ref=slots.at[pl.ds(d - 1, 1)],
                send_sem=send_sems.at[d - 1],
                recv_sem=recv_sems.at[d - 1],
                device_id=((my + d) % N_DEV,),
                device_id_type=pl.DeviceIdType.MESH,
            )
            rdmas.append(rdma)

            @pl.when(my + d < N_DEV)
            def _(rdma=rdma):
                rdma.start()

        scan = _local_cumprod(a, m, n)

        for d in range(1, N_DEV):

            @pl.when(d <= my)
            def _(rdma=rdmas[d - 1]):
                rdma.wait_recv()

        vals = slots[:, :]
        idx = lax.broadcasted_iota(jnp.int32, (N_DEV - 1, n), 0)
        vals = jnp.where(idx < my, vals, jnp.ones_like(vals))
        ones_row = jnp.ones((1, n), jnp.float32)
        prefix = _tree_prod(jnp.concatenate([vals, ones_row], axis=0), N_DEV)

        out_ref[:, :] = scan * prefix

        for d in range(1, N_DEV):

            @pl.when(my + d < N_DEV)
            def _(rdma=rdmas[d - 1]):
                rdma.wait_send()

    return pl.pallas_call(
        body,
        out_shape=jax.ShapeDtypeStruct((m, n), jnp.float32),
        in_specs=[pl.BlockSpec(memory_space=pltpu.VMEM)],
        out_specs=pl.BlockSpec(memory_space=pltpu.VMEM),
        scratch_shapes=[
            pltpu.VMEM((1, n), jnp.float32),
            pltpu.VMEM((N_DEV - 1, n), jnp.float32),
            pltpu.SemaphoreType.DMA((N_DEV - 1,)),
            pltpu.SemaphoreType.DMA((N_DEV - 1,)),
        ],
        compiler_params=pltpu.CompilerParams(collective_id=0),
    )(x)
